# Initial kernel scaffold; baseline (speedup 1.0000x reference)
import functools

import jax
import jax.numpy as jnp
from jax import lax
from jax.experimental import pallas as pl
from jax.experimental.pallas import tpu as pltpu
from jax.experimental.pallas import tpu_sc as plsc

N_CORES = 2
N_SUB = 16
CHUNK = 128
NBUF = 2


def _sc_scatter_gather(n_pad, d_feat, chunks_per_tile, rows_per_tile):
  mesh = plsc.VectorSubcoreMesh(core_axis_name="c", subcore_axis_name="s")

  def body(x_hbm, src_hbm, dst_hbm, zeros_hbm, out_hbm,
           src_v, dst_v, bufs_v, acc_sh, *sems):
    cid = lax.axis_index("c")
    sid = lax.axis_index("s")

    pltpu.sync_copy(src_hbm.at[cid, sid], src_v)
    pltpu.sync_copy(dst_hbm.at[cid, sid], dst_v)

    row0 = sid * rows_per_tile
    pltpu.sync_copy(zeros_hbm, acc_sh.at[pl.ds(row0, rows_per_tile)])
    plsc.subcore_barrier()

    def start_gather(c, b):
      pltpu.async_copy(x_hbm.at[src_v.at[c]], bufs_v.at[b], sems[b])

    for b in range(NBUF):
      start_gather(b, b)

    @pl.loop(0, chunks_per_tile // NBUF)
    def _outer(i):
      c0 = i * NBUF
      for b in range(NBUF):
        c = c0 + b
        pltpu.make_async_copy(
            x_hbm.at[src_v.at[c]], bufs_v.at[b], sems[b]).wait()
        pltpu.sync_copy(bufs_v.at[b], acc_sh.at[dst_v.at[c]], add=True)
        @pl.when(c + NBUF < chunks_per_tile)
        def _():
          start_gather(c + NBUF, b)

    plsc.subcore_barrier()
    pltpu.sync_copy(acc_sh.at[pl.ds(row0, rows_per_tile)],
                    out_hbm.at[cid, pl.ds(row0, rows_per_tile)])

  return pl.kernel(
      body,
      out_type=jax.ShapeDtypeStruct((N_CORES, n_pad, d_feat), jnp.float32),
      mesh=mesh,
      scratch_types=[
          pltpu.VMEM((chunks_per_tile, CHUNK), jnp.int32),
          pltpu.VMEM((chunks_per_tile, CHUNK), jnp.int32),
          pltpu.VMEM((NBUF, CHUNK, d_feat), jnp.float32),
          pltpu.VMEM_SHARED((n_pad, d_feat), jnp.float32),
      ] + [pltpu.SemaphoreType.DMA] * NBUF,
  )


def _combine(parts, n_nodes, block_rows):
  d_feat = parts.shape[2]
  grid = n_nodes // block_rows

  def body(p_ref, o_ref):
    o_ref[...] = p_ref[0] + p_ref[1]

  return pl.pallas_call(
      body,
      grid=(grid,),
      in_specs=[pl.BlockSpec((2, block_rows, d_feat), lambda i: (0, i, 0))],
      out_specs=pl.BlockSpec((block_rows, d_feat), lambda i: (i, 0)),
      out_shape=jax.ShapeDtypeStruct((n_nodes, d_feat), jnp.float32),
  )(parts)


def kernel(x, edge_index):
  n_nodes, d_feat = x.shape
  n_edges = edge_index.shape[1]

  src = edge_index[0].astype(jnp.int32)
  dst = edge_index[1].astype(jnp.int32)

  per_round = N_CORES * N_SUB * CHUNK
  chunks_per_tile = -(-n_edges // per_round)
  chunks_per_tile += chunks_per_tile % NBUF
  e_pad = N_CORES * N_SUB * chunks_per_tile * CHUNK

  n_pad = -(-(n_nodes + 1) // (N_SUB * 8)) * (N_SUB * 8)
  rows_per_tile = n_pad // N_SUB

  n_extra = e_pad - n_edges
  pad_dst = n_nodes + jnp.arange(n_extra, dtype=jnp.int32) % (n_pad - n_nodes)
  src = jnp.concatenate([src, jnp.zeros((n_extra,), jnp.int32)])
  dst = jnp.concatenate([dst, pad_dst])
  src = src.reshape(N_CORES, N_SUB, chunks_per_tile, CHUNK)
  dst = dst.reshape(N_CORES, N_SUB, chunks_per_tile, CHUNK)

  zeros = jnp.zeros((rows_per_tile, d_feat), jnp.float32)

  parts = _sc_scatter_gather(n_pad, d_feat, chunks_per_tile, rows_per_tile)(
      x, src, dst, zeros)

  block_rows = 1000 if n_nodes % 1000 == 0 else 8
  return _combine(parts, n_nodes, block_rows)


# speedup vs baseline: 3.5815x; 3.5815x over previous
import jax
import jax.numpy as jnp
from jax import lax
from jax.experimental import pallas as pl
from jax.experimental.pallas import tpu as pltpu
from jax.experimental.pallas import tpu_sc as plsc

N_CORES = 2
N_SUB = 16
CHUNK = 128
NBUF = 2


def _sc_scatter_gather(n_pad, d_feat, chunks_per_tile, rows_per_tile):
  mesh = plsc.VectorSubcoreMesh(core_axis_name="c", subcore_axis_name="s")

  def body(x_hbm, idx_hbm, zeros_hbm, out_hbm,
           idx_v, bufs_v, acc_sh, isem0, isem1, gsem0, gsem1):
    isems = (isem0, isem1)
    gsems = (gsem0, gsem1)
    cid = lax.axis_index("c")
    sid = lax.axis_index("s")

    row0 = sid * rows_per_tile
    pltpu.sync_copy(zeros_hbm, acc_sh.at[pl.ds(row0, rows_per_tile)])

    def idx_start(c, b):
      pltpu.async_copy(idx_hbm.at[cid, sid, c], idx_v.at[b], isems[b])

    def idx_wait(c, b):
      pltpu.make_async_copy(
          idx_hbm.at[cid, sid, c], idx_v.at[b], isems[b]).wait()

    def gather_start(c, b):
      pltpu.async_copy(x_hbm.at[idx_v.at[b, 0]], bufs_v.at[b], gsems[b])

    def gather_wait(c, b):
      pltpu.make_async_copy(
          x_hbm.at[idx_v.at[b, 0]], bufs_v.at[b], gsems[b]).wait()

    idx_start(0, 0)
    idx_start(1, 1)
    plsc.subcore_barrier()
    idx_wait(0, 0)
    gather_start(0, 0)

    @pl.loop(0, chunks_per_tile // NBUF)
    def _outer(i):
      c0 = i * NBUF
      for b in range(NBUF):
        c = c0 + b
        nb = (b + 1) % NBUF
        @pl.when(c + 1 < chunks_per_tile)
        def _():
          idx_wait(c + 1, nb)
          gather_start(c + 1, nb)
        gather_wait(c, b)
        pltpu.sync_copy(bufs_v.at[b], acc_sh.at[idx_v.at[b, 1]], add=True)
        @pl.when(c + NBUF < chunks_per_tile)
        def _():
          idx_start(c + NBUF, b)

    plsc.subcore_barrier()
    pltpu.sync_copy(acc_sh.at[pl.ds(row0, rows_per_tile)],
                    out_hbm.at[cid, pl.ds(row0, rows_per_tile)])

  return pl.kernel(
      body,
      out_type=jax.ShapeDtypeStruct((N_CORES, n_pad, d_feat), jnp.float32),
      mesh=mesh,
      scratch_types=[
          pltpu.VMEM((NBUF, 2, CHUNK), jnp.int32),
          pltpu.VMEM((NBUF, CHUNK, d_feat), jnp.float32),
          pltpu.VMEM_SHARED((n_pad, d_feat), jnp.float32),
          pltpu.SemaphoreType.DMA,
          pltpu.SemaphoreType.DMA,
          pltpu.SemaphoreType.DMA,
          pltpu.SemaphoreType.DMA,
      ],
  )


def _combine(parts, n_nodes, block_rows):
  d_feat = parts.shape[2]
  grid = n_nodes // block_rows

  def body(p_ref, o_ref):
    o_ref[...] = p_ref[0] + p_ref[1]

  return pl.pallas_call(
      body,
      grid=(grid,),
      in_specs=[pl.BlockSpec((2, block_rows, d_feat), lambda i: (0, i, 0))],
      out_specs=pl.BlockSpec((block_rows, d_feat), lambda i: (i, 0)),
      out_shape=jax.ShapeDtypeStruct((n_nodes, d_feat), jnp.float32),
  )(parts)


def kernel(x, edge_index):
  n_nodes, d_feat = x.shape
  n_edges = edge_index.shape[1]

  src = edge_index[0].astype(jnp.int32)
  dst = edge_index[1].astype(jnp.int32)

  per_round = N_CORES * N_SUB * CHUNK
  chunks_per_tile = -(-n_edges // per_round)
  chunks_per_tile += chunks_per_tile % NBUF
  e_pad = N_CORES * N_SUB * chunks_per_tile * CHUNK

  n_pad = -(-(n_nodes + 1) // (N_SUB * 8)) * (N_SUB * 8)
  rows_per_tile = n_pad // N_SUB

  n_extra = e_pad - n_edges
  pad_dst = n_nodes + jnp.arange(n_extra, dtype=jnp.int32) % (n_pad - n_nodes)
  src = jnp.concatenate([src, jnp.zeros((n_extra,), jnp.int32)])
  dst = jnp.concatenate([dst, pad_dst])
  idx = jnp.stack([
      src.reshape(N_CORES, N_SUB, chunks_per_tile, CHUNK),
      dst.reshape(N_CORES, N_SUB, chunks_per_tile, CHUNK),
  ], axis=3)

  zeros = jnp.zeros((rows_per_tile, d_feat), jnp.float32)

  parts = _sc_scatter_gather(n_pad, d_feat, chunks_per_tile, rows_per_tile)(
      x, idx, zeros)

  block_rows = 1000 if n_nodes % 1000 == 0 else 8
  return _combine(parts, n_nodes, block_rows)
